# baseline (speedup 1.0000x reference)
import functools

import jax
import jax.numpy as jnp
from jax import lax
from jax.experimental import pallas as pl
from jax.experimental.pallas import tpu as pltpu
from jax.experimental.pallas import tpu_sc as plsc

D = 64
SEQ = 2048
B_TOTAL = 8192
NC, NS, L = 2, 16, 16
NW = NC * NS
BPW = B_TOTAL // NW
GRP = 4
NGRP = BPW // GRP
NSG = BPW // L
NBUF = 12

_mesh = plsc.VectorSubcoreMesh(core_axis_name="c", subcore_axis_name="s")


@functools.partial(
    pl.kernel,
    mesh=_mesh,
    out_type=jax.ShapeDtypeStruct((D, B_TOTAL), jnp.float32),
    scratch_types=[
        pltpu.VMEM((8, 128), jnp.int32),
        pltpu.VMEM((NBUF, D, 128), jnp.float32),
        pltpu.VMEM((2, D, 128), jnp.float32),
        pltpu.SemaphoreType.DMA,
        pltpu.SemaphoreType.DMA,
    ],
    compiler_params=pltpu.CompilerParams(needs_layout_passes=False),
)
def _emb_lookup(idx_hbm, tok_hbm, pos_hbm, out_hbm,
                idx_v, stage, acc, sem, psem):
    wid = lax.axis_index("s") * NC + lax.axis_index("c")
    base = pl.multiple_of(wid * BPW, BPW)
    pos_base = pl.multiple_of(lax.rem(base, SEQ), BPW)

    blk0 = pl.multiple_of((wid // 4) * 8, 8)
    pltpu.sync_copy(idx_hbm.at[pl.ds(blk0, 8)], idx_v)
    pos_cps = [
        pltpu.async_copy(
            pos_hbm.at[:, pl.ds(pos_base + h * 128, 128)], acc.at[h], psem
        )
        for h in range(2)
    ]

    row0 = lax.rem(wid, 4) * 2
    lanes = lax.iota(jnp.int32, L)

    def load_iv(sg):
        sgc = lax.min(sg, NSG - 1)
        return idx_v[row0 + sgc // 8, pl.ds(lax.rem(sgc, 8) * L, L)]

    def fire(cv, l, r):
        col = pl.multiple_of(lax.shift_right_logical(cv[l], 7) * 128, 128)
        slot = lax.rem(r, NBUF)
        for i in range(D // 8):
            pltpu.async_copy(
                tok_hbm.at[pl.ds(i * 8, 8), pl.ds(col, 128)],
                stage.at[slot, pl.ds(i * 8, 8)],
                sem,
            )

    def fire_group(cv, lb, g):
        if isinstance(g, int):
            for l in range(GRP):
                fire(cv, lb + l, g * GRP + l)
            return

        @pl.when(g < NGRP)
        def _():
            for l in range(GRP):
                fire(cv, lb + l, g * GRP + l)

    def select(pv, l, r):
        slot = lax.rem(r, NBUF)
        for i in range(D // 8):
            pltpu.make_async_copy(
                tok_hbm.at[pl.ds(0, 8), pl.ds(0, 128)],
                stage.at[slot, pl.ds(i * 8, 8)],
                sem,
            ).wait()
        pvec = jnp.full((L,), lax.bitwise_and(pv[l], 127), jnp.int32)
        rvec = jnp.full((L,), lax.rem(r, 128), jnp.int32)
        buf = stage.at[slot]
        half = acc.at[r // 128]
        for q in range(D // L):
            dvec = lanes + (q * L)
            vals = plsc.load_gather(buf, [dvec, pvec])
            plsc.addupdate_scatter(half, [dvec, rvec], vals)

    iv0 = load_iv(0)
    for g in range(3):
        fire_group(iv0, g * GRP, g)
    for cp in pos_cps:
        cp.wait()

    def step(sg, iv):
        iv_next = load_iv(sg + 1)
        for j in range(4):
            g_sel = sg * 4 + j
            for l in range(GRP):
                select(iv, j * GRP + l, g_sel * GRP + l)
            if j == 0:
                fire_group(iv, 3 * GRP, g_sel + 3)
            else:
                fire_group(iv_next, (j - 1) * GRP, g_sel + 3)
        return iv_next

    lax.fori_loop(0, NSG, step, iv0)

    for h in range(2):
        pltpu.sync_copy(acc.at[h], out_hbm.at[:, pl.ds(base + h * 128, 128)])


def kernel(x, token_table, pos_table):
    batch, seq = x.shape
    idx = x.astype(jnp.int32).reshape(NW * 2, 128)
    out_t = _emb_lookup(idx, token_table.T, pos_table.T)
    return out_t.T.reshape(batch, seq, D)


# speedup vs baseline: 1.0126x; 1.0126x over previous
import functools

import jax
import jax.numpy as jnp
from jax import lax
from jax.experimental import pallas as pl
from jax.experimental.pallas import tpu as pltpu
from jax.experimental.pallas import tpu_sc as plsc

D = 64
SEQ = 2048
B_TOTAL = 8192
NC, NS, L = 2, 16, 16
NW = NC * NS
BPW = B_TOTAL // NW
GRP = 4
NGRP = BPW // GRP
NSG = BPW // L
NBUF = 12

_mesh = plsc.VectorSubcoreMesh(core_axis_name="c", subcore_axis_name="s")


@functools.partial(
    pl.kernel,
    mesh=_mesh,
    out_type=jax.ShapeDtypeStruct((D, B_TOTAL), jnp.float32),
    scratch_types=[
        pltpu.VMEM((8, 128), jnp.int32),
        pltpu.VMEM((NBUF, D, 128), jnp.float32),
        pltpu.VMEM((2, D, 128), jnp.float32),
        pltpu.SemaphoreType.DMA,
        pltpu.SemaphoreType.DMA,
    ],
    compiler_params=pltpu.CompilerParams(needs_layout_passes=False),
)
def _emb_lookup(idx_hbm, tok_hbm, pos_hbm, out_hbm,
                idx_v, stage, acc, sem, psem):
    wid = lax.axis_index("s") * NC + lax.axis_index("c")
    base = pl.multiple_of(wid * BPW, BPW)
    pos_base = pl.multiple_of(lax.rem(base, SEQ), BPW)

    blk0 = pl.multiple_of((wid // 4) * 8, 8)
    pltpu.sync_copy(idx_hbm.at[pl.ds(blk0, 8)], idx_v)
    pos_cps = [
        pltpu.async_copy(
            pos_hbm.at[:, pl.ds(pos_base + h * 128, 128)], acc.at[h], psem
        )
        for h in range(2)
    ]

    row0 = lax.rem(wid, 4) * 2
    lanes = lax.iota(jnp.int32, L)

    def load_iv(sg):
        sgc = lax.min(sg, NSG - 1)
        return idx_v[row0 + sgc // 8, pl.ds(lax.rem(sgc, 8) * L, L)]

    def fire(cv, l, r):
        col = pl.multiple_of(lax.shift_right_logical(cv[l], 7) * 128, 128)
        pltpu.async_copy(
            tok_hbm.at[:, pl.ds(col, 128)], stage.at[lax.rem(r, NBUF)], sem
        )

    def fire_group(cv, lb, g):
        if isinstance(g, int):
            for l in range(GRP):
                fire(cv, lb + l, g * GRP + l)
            return

        @pl.when(g < NGRP)
        def _():
            for l in range(GRP):
                fire(cv, lb + l, g * GRP + l)

    def select(pv, l, r):
        slot = lax.rem(r, NBUF)
        pltpu.make_async_copy(
            tok_hbm.at[:, pl.ds(0, 128)], stage.at[slot], sem
        ).wait()
        pvec = jnp.full((L,), lax.bitwise_and(pv[l], 127), jnp.int32)
        rvec = jnp.full((L,), lax.rem(r, 128), jnp.int32)
        buf = stage.at[slot]
        half = acc.at[r // 128]
        for q in range(D // L):
            dvec = lanes + (q * L)
            vals = plsc.load_gather(buf, [dvec, pvec])
            plsc.addupdate_scatter(half, [dvec, rvec], vals)

    iv0 = load_iv(0)
    for g in range(3):
        fire_group(iv0, g * GRP, g)
    for cp in pos_cps:
        cp.wait()

    def step(sg, iv):
        iv_next = load_iv(sg + 1)
        for j in range(4):
            g_sel = sg * 4 + j
            for l in range(GRP):
                select(iv, j * GRP + l, g_sel * GRP + l)
            if j == 0:
                fire_group(iv, 3 * GRP, g_sel + 3)
            else:
                fire_group(iv_next, (j - 1) * GRP, g_sel + 3)
        return iv_next

    lax.fori_loop(0, NSG, step, iv0)

    for h in range(2):
        pltpu.sync_copy(acc.at[h], out_hbm.at[:, pl.ds(base + h * 128, 128)])


def kernel(x, token_table, pos_table):
    batch, seq = x.shape
    idx = x.astype(jnp.int32).reshape(NW * 2, 128)
    out_t = _emb_lookup(idx, token_table.T, pos_table.T)
    return out_t.T.reshape(batch, seq, D)
